# baseline (speedup 1.0000x reference)
import functools

import jax
import jax.numpy as jnp
from jax import lax
from jax.experimental import pallas as pl
from jax.experimental.pallas import tpu as pltpu
from jax.experimental.pallas import tpu_sc as plsc

NUM_LOCATION = 100000
EMBED_DIM = 32
BATCH = 16384
N_COLS = 4

_INFO = plsc.get_sparse_core_info()
NUM_CORES = _INFO.num_cores
NUM_SUBCORES = _INFO.num_subcores
LANES = _INFO.num_lanes
NUM_WORKERS = NUM_CORES * NUM_SUBCORES

CHUNK = 1024
N_CHUNKS = BATCH // CHUNK


def _body(uft_hbm, tt_hbm, out_hbm, idx_v, row_v, out_v,
          sem_in, sem_out0, sem_out1):
    c = lax.axis_index("s") * NUM_CORES + lax.axis_index("c")

    idx_cp = pltpu.async_copy(uft_hbm.at[1], idx_v, sem_in)
    row_cp = pltpu.async_copy(tt_hbm.at[c], row_v, sem_in)
    idx_cp.wait()
    row_cp.wait()

    sems = (sem_out0, sem_out1)
    pending = [None, None]
    for ci in range(N_CHUNKS):
        buf = ci % 2
        if pending[buf] is not None:
            pending[buf].wait()
        for k in range(CHUNK // LANES):
            idx16 = idx_v[pl.ds(ci * CHUNK + k * LANES, LANES)]
            out_v[buf, pl.ds(k * LANES, LANES)] = plsc.load_gather(
                row_v, [idx16])
        pending[buf] = pltpu.async_copy(
            out_v.at[buf], out_hbm.at[c, pl.ds(ci * CHUNK, CHUNK)],
            sems[buf])
    pending[0].wait()
    pending[1].wait()


@jax.jit
def kernel(user_fea, embedding_location):
    mesh = plsc.VectorSubcoreMesh(core_axis_name="c", subcore_axis_name="s")
    run = functools.partial(
        pl.kernel,
        out_type=jax.ShapeDtypeStruct((EMBED_DIM, BATCH), jnp.float32),
        mesh=mesh,
        compiler_params=pltpu.CompilerParams(needs_layout_passes=False),
        scratch_types=[
            pltpu.VMEM((BATCH,), jnp.int32),
            pltpu.VMEM((NUM_LOCATION,), jnp.float32),
            pltpu.VMEM((2, CHUNK), jnp.float32),
            pltpu.SemaphoreType.DMA,
            pltpu.SemaphoreType.DMA,
            pltpu.SemaphoreType.DMA,
        ],
    )(_body)
    out_t = run(user_fea.T, embedding_location.T)
    return out_t.T


# speedup vs baseline: 1.2969x; 1.2969x over previous
import functools

import jax
import jax.numpy as jnp
from jax import lax
from jax.experimental import pallas as pl
from jax.experimental.pallas import tpu as pltpu
from jax.experimental.pallas import tpu_sc as plsc

NUM_LOCATION = 100000
EMBED_DIM = 32
BATCH = 16384
N_COLS = 4

_INFO = plsc.get_sparse_core_info()
NUM_CORES = _INFO.num_cores
NUM_SUBCORES = _INFO.num_subcores
LANES = _INFO.num_lanes
NUM_WORKERS = NUM_CORES * NUM_SUBCORES

CHUNK = 1024
N_CHUNKS = BATCH // CHUNK


def _body(uft_hbm, tt_hbm, out_hbm, idx_v, row_v, out_v,
          sem_in, sem_out0, sem_out1):
    c = lax.axis_index("s") * NUM_CORES + lax.axis_index("c")

    idx_cp = pltpu.async_copy(uft_hbm.at[1], idx_v, sem_in)
    row_cp = pltpu.async_copy(tt_hbm.at[c], row_v, sem_in)
    idx_cp.wait()
    row_cp.wait()

    sems = (sem_out0, sem_out1)
    pending = [None, None]
    for ci in range(N_CHUNKS):
        buf = ci % 2
        if pending[buf] is not None:
            pending[buf].wait()

        @plsc.parallel_loop(0, CHUNK // LANES, unroll=8)
        def _gather(k):
            idx16 = idx_v[pl.ds(ci * CHUNK + k * LANES, LANES)]
            out_v[buf, pl.ds(k * LANES, LANES)] = plsc.load_gather(
                row_v, [idx16])

        pending[buf] = pltpu.async_copy(
            out_v.at[buf], out_hbm.at[c, pl.ds(ci * CHUNK, CHUNK)],
            sems[buf])
    pending[0].wait()
    pending[1].wait()


@jax.jit
def kernel(user_fea, embedding_location):
    mesh = plsc.VectorSubcoreMesh(core_axis_name="c", subcore_axis_name="s")
    run = functools.partial(
        pl.kernel,
        out_type=jax.ShapeDtypeStruct((EMBED_DIM, BATCH), jnp.float32),
        mesh=mesh,
        compiler_params=pltpu.CompilerParams(needs_layout_passes=False),
        scratch_types=[
            pltpu.VMEM((BATCH,), jnp.int32),
            pltpu.VMEM((NUM_LOCATION,), jnp.float32),
            pltpu.VMEM((2, CHUNK), jnp.float32),
            pltpu.SemaphoreType.DMA,
            pltpu.SemaphoreType.DMA,
            pltpu.SemaphoreType.DMA,
        ],
    )(_body)
    out_t = run(user_fea.T, embedding_location.T)
    return out_t.T


# speedup vs baseline: 1.3625x; 1.0506x over previous
import functools

import jax
import jax.numpy as jnp
from jax import lax
from jax.experimental import pallas as pl
from jax.experimental.pallas import tpu as pltpu
from jax.experimental.pallas import tpu_sc as plsc

NUM_LOCATION = 100000
EMBED_DIM = 32
BATCH = 16384
N_COLS = 4

_INFO = plsc.get_sparse_core_info()
NUM_CORES = _INFO.num_cores
NUM_SUBCORES = _INFO.num_subcores
LANES = _INFO.num_lanes
NUM_WORKERS = NUM_CORES * NUM_SUBCORES

CHUNK = 4096
N_CHUNKS = BATCH // CHUNK


def _body(uft_hbm, tt_hbm, out_hbm, idx_v, row_v, out_v,
          sem_in, sem_out0, sem_out1):
    c = lax.axis_index("s") * NUM_CORES + lax.axis_index("c")

    idx_cp = pltpu.async_copy(uft_hbm.at[1], idx_v, sem_in)
    row_cp = pltpu.async_copy(tt_hbm.at[c], row_v, sem_in)
    idx_cp.wait()
    row_cp.wait()

    sems = (sem_out0, sem_out1)
    pending = [None, None]
    for ci in range(N_CHUNKS):
        buf = ci % 2
        if pending[buf] is not None:
            pending[buf].wait()

        @plsc.parallel_loop(0, CHUNK // LANES, unroll=8)
        def _gather(k):
            idx16 = idx_v[pl.ds(ci * CHUNK + k * LANES, LANES)]
            out_v[buf, pl.ds(k * LANES, LANES)] = plsc.load_gather(
                row_v, [idx16])

        pending[buf] = pltpu.async_copy(
            out_v.at[buf], out_hbm.at[c, pl.ds(ci * CHUNK, CHUNK)],
            sems[buf])
    pending[0].wait()
    pending[1].wait()


@jax.jit
def kernel(user_fea, embedding_location):
    mesh = plsc.VectorSubcoreMesh(core_axis_name="c", subcore_axis_name="s")
    run = functools.partial(
        pl.kernel,
        out_type=jax.ShapeDtypeStruct((EMBED_DIM, BATCH), jnp.float32),
        mesh=mesh,
        compiler_params=pltpu.CompilerParams(needs_layout_passes=False),
        scratch_types=[
            pltpu.VMEM((BATCH,), jnp.int32),
            pltpu.VMEM((NUM_LOCATION,), jnp.float32),
            pltpu.VMEM((2, CHUNK), jnp.float32),
            pltpu.SemaphoreType.DMA,
            pltpu.SemaphoreType.DMA,
            pltpu.SemaphoreType.DMA,
        ],
    )(_body)
    out_t = run(user_fea.T, embedding_location.T)
    return out_t.T
